# Initial kernel scaffold; baseline (speedup 1.0000x reference)
import jax
import jax.numpy as jnp
from jax.experimental import pallas as pl


def kernel(h_ns, h_nv, h_nt, h_es, h_ev, atom_index1, atom_index2, Wns1, bns1, Wns2, bns2, Wes1, bes1, Wes2, bes2, Wnv1, Wnv2, Wnt1, Wnt2, Wev1, Wc1, bc1, Wc2, bc2):
    raise NotImplementedError("write your pallas kernel here")


# speedup vs baseline: 6.6210x; 6.6210x over previous
import functools

import jax
import jax.numpy as jnp
from jax import lax
from jax.experimental import pallas as pl
from jax.experimental.pallas import tpu as pltpu
from jax.experimental.pallas import tpu_sc as plsc

_N = 50000
_E = 800000

_NC = 2
_NT = 16
_W = 640
_NWIN = _E // _W
_RPT = 3128
_ACCW = _RPT * 20
_ACCW_LAST = (_N - 15 * _RPT) * 20

_BE = 8000
_BN = 2000


def _silu(x):
    return x * (1.0 / (1.0 + jnp.exp(-x)))


def _sigmoid(x):
    return 1.0 / (1.0 + jnp.exp(-x))


def _edge_body(hes, hev, wes1, bes1, wes2, bes2, wev, out):
    s = _silu(jnp.dot(hes[...], wes1[...]) + bes1[...])
    s = jnp.dot(s, wes2[...]) + bes2[...]
    v = jnp.dot(hev[...], wev[...])
    out[0] = s[:, 0:20]
    out[1] = jnp.concatenate([s[:, 20:32], v[:, 0:8]], axis=-1)
    out[2] = v[:, 8:28]
    out[3] = v[:, 28:48]


def _edge_payload(h_es, h_ev_flat, Wes1T, bes1, Wes2T, bes2, WevkT):
    grid = (_E // _BE,)
    return pl.pallas_call(
        _edge_body,
        grid=grid,
        in_specs=[
            pl.BlockSpec((_BE, 64), lambda i: (i, 0)),
            pl.BlockSpec((_BE, 48), lambda i: (i, 0)),
            pl.BlockSpec((64, 32), lambda i: (0, 0)),
            pl.BlockSpec((1, 32), lambda i: (0, 0)),
            pl.BlockSpec((32, 32), lambda i: (0, 0)),
            pl.BlockSpec((1, 32), lambda i: (0, 0)),
            pl.BlockSpec((48, 48), lambda i: (0, 0)),
        ],
        out_specs=pl.BlockSpec((4, _BE, 20), lambda i: (0, i, 0)),
        out_shape=jax.ShapeDtypeStruct((4, _E, 20), jnp.float32),
        compiler_params=pltpu.CompilerParams(
            dimension_semantics=("arbitrary",)),
    )(h_es, h_ev_flat, Wes1T, bes1, Wes2T, bes2, WevkT)


def _init_body(hns, hnv, wns1, bns1, wns2, bns2, wnv1, out):
    s = _silu(jnp.dot(hns[...], wns1[...]) + bns1[...])
    s = jnp.dot(s, wns2[...]) + bns2[...]
    v = jnp.dot(hnv[...], wnv1[...])
    out[0] = s[:, 0:20]
    out[1] = jnp.concatenate([s[:, 20:32], v[:, 0:8]], axis=-1)
    out[2] = v[:, 8:28]
    out[3] = v[:, 28:48]


def _node_init(h_ns, h_nv_flat, Wns1T, bns1, Wns2T, bns2, Wnv1kT):
    grid = (_N // _BN,)
    return pl.pallas_call(
        _init_body,
        grid=grid,
        in_specs=[
            pl.BlockSpec((_BN, 64), lambda i: (i, 0)),
            pl.BlockSpec((_BN, 48), lambda i: (i, 0)),
            pl.BlockSpec((64, 32), lambda i: (0, 0)),
            pl.BlockSpec((1, 32), lambda i: (0, 0)),
            pl.BlockSpec((32, 32), lambda i: (0, 0)),
            pl.BlockSpec((1, 32), lambda i: (0, 0)),
            pl.BlockSpec((48, 48), lambda i: (0, 0)),
        ],
        out_specs=pl.BlockSpec((4, _BN, 20), lambda i: (0, i, 0)),
        out_shape=jax.ShapeDtypeStruct((4, _N, 20), jnp.float32),
        compiler_params=pltpu.CompilerParams(
            dimension_semantics=("arbitrary",)),
    )(h_ns, h_nv_flat, Wns1T, bns1, Wns2T, bns2, Wnv1kT)


def _sc_body(P, idx1, idx2, init, A, pbuf, acc, ib1, ib2):
    c = lax.axis_index("c")
    s = lax.axis_index("s")
    lane = lax.iota(jnp.int32, 16)
    tailmask = lane >= 12
    lo = s * _RPT
    hi = jnp.minimum(lo + _RPT, _N)

    for k in range(2):
        p = 2 * k + c

        @pl.when(s < _NT - 1)
        def _():
            pltpu.sync_copy(init.at[pl.ds(p * (_N * 20) + lo * 20, _ACCW)],
                            acc.at[pl.ds(0, _ACCW)])

        @pl.when(s == _NT - 1)
        def _():
            pltpu.sync_copy(
                init.at[pl.ds(p * (_N * 20) + lo * 20, _ACCW_LAST)],
                acc.at[pl.ds(0, _ACCW_LAST)])

        def win_body(w, carry):
            e0 = w * _W
            pltpu.sync_copy(
                P.at[pl.ds(p * (_E * 20) + e0 * 20, _W * 20)], pbuf)
            pltpu.sync_copy(idx1.at[pl.ds(e0, _W)], ib1)
            pltpu.sync_copy(idx2.at[pl.ds(e0, _W)], ib2)

            def grp_body(g, cc):
                eoff = (g * 16 + lane) * 20
                for ib in (ib1, ib2):
                    iv = ib[pl.ds(g * 16, 16)]
                    m = (iv >= lo) & (iv < hi)
                    base = jnp.where(m, (iv - lo) * 20, 0)
                    for col in range(20):
                        vals = plsc.load_gather(pbuf, [eoff + col])
                        plsc.addupdate_scatter(acc, [base + col], vals,
                                               mask=m)
                return cc

            lax.fori_loop(0, _W // 16, grp_body, 0)
            return carry

        lax.fori_loop(0, _NWIN, win_body, 0)

        @pl.when(s < _NT - 1)
        def _():
            pltpu.sync_copy(acc.at[pl.ds(0, _ACCW)],
                            A.at[pl.ds(p * (_N * 20) + lo * 20, _ACCW)])

        @pl.when(s == _NT - 1)
        def _():
            pltpu.sync_copy(
                acc.at[pl.ds(0, _ACCW_LAST)],
                A.at[pl.ds(p * (_N * 20) + lo * 20, _ACCW_LAST)])


def _sc_scatter(P_flat, idx1, idx2, init_flat):
    mesh = plsc.VectorSubcoreMesh(core_axis_name="c", subcore_axis_name="s",
                                  num_cores=_NC, num_subcores=_NT)
    fn = functools.partial(
        pl.kernel,
        out_type=jax.ShapeDtypeStruct((4 * _N * 20,), jnp.float32),
        mesh=mesh,
        compiler_params=pltpu.CompilerParams(needs_layout_passes=False),
        scratch_types=[
            pltpu.VMEM((_W * 20,), jnp.float32),
            pltpu.VMEM((_ACCW,), jnp.float32),
            pltpu.VMEM((_W,), jnp.int32),
            pltpu.VMEM((_W,), jnp.int32),
        ],
    )(_sc_body)
    return fn(P_flat, idx1, idx2, init_flat)


def _finish_body(a, hns, hnv, hnt, wnt1, wc1ns, wc1nv, wc1nt, bc1,
                 wc2ns, bc2ns, wc2nv, bc2nv, wc2nt, bc2nt, wnv2, wnt2,
                 ns_out, nv_out, nt_out):
    af = jnp.concatenate([a[0], a[1], a[2], a[3]], axis=-1)
    ns_int = af[:, :32]
    nv_int = af[:, 32:]
    sq = nv_int * nv_int
    nv_norm = jnp.sqrt(sq[:, 0:16] + sq[:, 16:32] + sq[:, 32:48])
    nt1 = jnp.dot(hnt[...], wnt1[...])
    sqt = nt1 * nt1
    acc = sqt[:, 0:16]
    for k in range(1, 9):
        acc = acc + sqt[:, 16 * k:16 * (k + 1)]
    nt_norm = jnp.sqrt(acc)
    h1 = _silu(jnp.dot(ns_int, wc1ns[...]) + jnp.dot(nv_norm, wc1nv[...])
               + jnp.dot(nt_norm, wc1nt[...]) + bc1[...])
    ns_out[...] = jnp.dot(h1, wc2ns[...]) + bc2ns[...] + hns[...]
    gv = _sigmoid(jnp.dot(h1, wc2nv[...]) + bc2nv[...])
    gt = _sigmoid(jnp.dot(h1, wc2nt[...]) + bc2nt[...])
    nv2 = jnp.dot(hnv[...], wnv2[...])
    nv_out[...] = jnp.concatenate([gv, gv, gv], axis=-1) * nv2 + hnv[...]
    nt2 = jnp.dot(hnt[...], wnt2[...])
    gt9 = jnp.concatenate([gt] * 9, axis=-1)
    nt_out[...] = gt9 * nt2 + hnt[...]


def _node_finish(A, h_ns, h_nv_flat, h_nt_flat, Wnt1kT, Wc1Tns, Wc1Tnv,
                 Wc1Tnt, bc1, Wc2Tns, bc2ns, Wc2Tnv, bc2nv, Wc2Tnt, bc2nt,
                 Wnv2kT, Wnt2kT):
    grid = (_N // _BN,)
    full = lambda shape: pl.BlockSpec(shape, lambda i: tuple(0 for _ in shape))
    return pl.pallas_call(
        _finish_body,
        grid=grid,
        in_specs=[
            pl.BlockSpec((4, _BN, 20), lambda i: (0, i, 0)),
            pl.BlockSpec((_BN, 64), lambda i: (i, 0)),
            pl.BlockSpec((_BN, 48), lambda i: (i, 0)),
            pl.BlockSpec((_BN, 144), lambda i: (i, 0)),
            full((144, 144)),
            full((32, 32)),
            full((16, 32)),
            full((16, 32)),
            full((1, 32)),
            full((32, 64)),
            full((1, 64)),
            full((32, 16)),
            full((1, 16)),
            full((32, 16)),
            full((1, 16)),
            full((48, 48)),
            full((144, 144)),
        ],
        out_specs=[
            pl.BlockSpec((_BN, 64), lambda i: (i, 0)),
            pl.BlockSpec((_BN, 48), lambda i: (i, 0)),
            pl.BlockSpec((_BN, 144), lambda i: (i, 0)),
        ],
        out_shape=[
            jax.ShapeDtypeStruct((_N, 64), jnp.float32),
            jax.ShapeDtypeStruct((_N, 48), jnp.float32),
            jax.ShapeDtypeStruct((_N, 144), jnp.float32),
        ],
        compiler_params=pltpu.CompilerParams(
            dimension_semantics=("arbitrary",)),
    )(A, h_ns, h_nv_flat, h_nt_flat, Wnt1kT, Wc1Tns, Wc1Tnv, Wc1Tnt, bc1,
      Wc2Tns, bc2ns, Wc2Tnv, bc2nv, Wc2Tnt, bc2nt, Wnv2kT, Wnt2kT)


def kernel(h_ns, h_nv, h_nt, h_es, h_ev, atom_index1, atom_index2,
           Wns1, bns1, Wns2, bns2, Wes1, bes1, Wes2, bes2,
           Wnv1, Wnv2, Wnt1, Wnt2, Wev1, Wc1, bc1, Wc2, bc2):
    f32 = jnp.float32
    h_ev_flat = h_ev.reshape(_E, 48)
    h_nv_flat = h_nv.reshape(_N, 48)
    h_nt_flat = h_nt.reshape(_N, 144)
    i3 = jnp.eye(3, dtype=f32)
    i9 = jnp.eye(9, dtype=f32)
    WevkT = jnp.kron(i3, Wev1).T
    Wnv1kT = jnp.kron(i3, Wnv1).T
    Wnv2kT = jnp.kron(i3, Wnv2).T
    Wnt1kT = jnp.kron(i9, Wnt1).T
    Wnt2kT = jnp.kron(i9, Wnt2).T
    Wc1T = Wc1.T
    Wc2T = Wc2.T

    P = _edge_payload(h_es, h_ev_flat, Wes1.T, bes1.reshape(1, 32),
                      Wes2.T, bes2.reshape(1, 32), WevkT)
    INIT = _node_init(h_ns, h_nv_flat, Wns1.T, bns1.reshape(1, 32),
                      Wns2.T, bns2.reshape(1, 32), Wnv1kT)
    idx1 = atom_index1.astype(jnp.int32)
    idx2 = atom_index2.astype(jnp.int32)
    A_flat = _sc_scatter(P.reshape(4 * _E * 20), idx1, idx2,
                         INIT.reshape(4 * _N * 20))
    A = A_flat.reshape(4, _N, 20)
    ns_out, nv_out_flat, nt_out_flat = _node_finish(
        A, h_ns, h_nv_flat, h_nt_flat, Wnt1kT,
        Wc1T[0:32], Wc1T[32:48], Wc1T[48:64], bc1.reshape(1, 32),
        Wc2T[:, 0:64], bc2[0:64].reshape(1, 64),
        Wc2T[:, 64:80], bc2[64:80].reshape(1, 16),
        Wc2T[:, 80:96], bc2[80:96].reshape(1, 16),
        Wnv2kT, Wnt2kT)
    return (ns_out, nv_out_flat.reshape(_N, 3, 16),
            nt_out_flat.reshape(_N, 3, 3, 16))


# speedup vs baseline: 6.7660x; 1.0219x over previous
import functools

import jax
import jax.numpy as jnp
from jax import lax
from jax.experimental import pallas as pl
from jax.experimental.pallas import tpu as pltpu
from jax.experimental.pallas import tpu_sc as plsc

_N = 50000
_E = 800000

_NC = 2
_NT = 16
_W = 640
_NWIN = _E // _W
_RPT = 3128
_PS = 21
_ACCW = _RPT * _PS + 16
_ZITER = _ACCW // 16
_WCH = 640
_TAIL_LO = _RPT - 4 * _WCH
_TAIL_HI = (_N - 15 * _RPT) - 4 * _WCH

_BE = 8000
_BN = 2000


def _silu(x):
    return x * (1.0 / (1.0 + jnp.exp(-x)))


def _sigmoid(x):
    return 1.0 / (1.0 + jnp.exp(-x))


def _edge_body(hes, hev, wes1, bes1, wes2, bes2, wev, out):
    s = _silu(jnp.dot(hes[...], wes1[...]) + bes1[...])
    s = jnp.dot(s, wes2[...]) + bes2[...]
    v = jnp.dot(hev[...], wev[...])
    z = jnp.zeros_like(s[:, 0:1])
    out[0] = jnp.concatenate([s[:, 0:20], z], axis=-1)
    out[1] = jnp.concatenate([s[:, 20:32], v[:, 0:8], z], axis=-1)
    out[2] = jnp.concatenate([v[:, 8:28], z], axis=-1)
    out[3] = jnp.concatenate([v[:, 28:48], z], axis=-1)


def _edge_payload(h_es, h_ev_flat, Wes1T, bes1, Wes2T, bes2, WevkT):
    grid = (_E // _BE,)
    return pl.pallas_call(
        _edge_body,
        grid=grid,
        in_specs=[
            pl.BlockSpec((_BE, 64), lambda i: (i, 0)),
            pl.BlockSpec((_BE, 48), lambda i: (i, 0)),
            pl.BlockSpec((64, 32), lambda i: (0, 0)),
            pl.BlockSpec((1, 32), lambda i: (0, 0)),
            pl.BlockSpec((32, 32), lambda i: (0, 0)),
            pl.BlockSpec((1, 32), lambda i: (0, 0)),
            pl.BlockSpec((48, 48), lambda i: (0, 0)),
        ],
        out_specs=pl.BlockSpec((4, _BE, _PS), lambda i: (0, i, 0)),
        out_shape=jax.ShapeDtypeStruct((4, _E, _PS), jnp.float32),
        compiler_params=pltpu.CompilerParams(
            dimension_semantics=("arbitrary",)),
    )(h_es, h_ev_flat, Wes1T, bes1, Wes2T, bes2, WevkT)


def _sc_body(P, idx1, idx2, A, pbuf, acc, ib1, ib2):
    c = lax.axis_index("c")
    s = lax.axis_index("s")
    lane = lax.iota(jnp.int32, 16)
    zero16 = jnp.zeros((16,), jnp.float32)
    lo = s * _RPT
    hi = jnp.minimum(lo + _RPT, _N)

    for k in range(2):
        p = 2 * k + c

        def zero_body(i, cc):
            acc[pl.ds(i * 16, 16)] = zero16
            return cc

        lax.fori_loop(0, _ZITER, zero_body, 0)

        def win_body(w, carry):
            e0 = w * _W
            pltpu.sync_copy(
                P.at[pl.ds(p * (_E * _PS) + e0 * _PS, _W * _PS)], pbuf)
            pltpu.sync_copy(idx1.at[pl.ds(e0, _W)], ib1)
            pltpu.sync_copy(idx2.at[pl.ds(e0, _W)], ib2)

            def grp_body(g, cc):
                eoff = (g * 16 + lane) * _PS
                for ib in (ib1, ib2):
                    iv = ib[pl.ds(g * 16, 16)]
                    m = (iv >= lo) & (iv < hi)
                    base = jnp.where(m, (iv - lo) * _PS, 0)
                    for col in range(20):
                        vals = plsc.load_gather(pbuf, [eoff + col])
                        plsc.addupdate_scatter(acc, [base + col], vals,
                                               mask=m)
                return cc

            lax.fori_loop(0, _W // 16, grp_body, 0)
            return carry

        lax.fori_loop(0, _NWIN, win_body, 0)

        def _emit_chunk(j, rows):
            def rs_body(i, cc):
                v = i * 16 + lane
                q = v // 20
                r = v - q * 20
                vals = plsc.load_gather(acc, [(j * _WCH + q) * _PS + r])
                pbuf[pl.ds(i * 16, 16)] = vals
                return cc

            lax.fori_loop(0, rows * 20 // 16, rs_body, 0)
            pltpu.sync_copy(
                pbuf.at[pl.ds(0, rows * 20)],
                A.at[pl.ds(p * (_N * 20) + (lo + j * _WCH) * 20, rows * 20)])

        for j in range(4):
            _emit_chunk(j, _WCH)

        @pl.when(s < _NT - 1)
        def _():
            _emit_chunk(4, _TAIL_LO)

        @pl.when(s == _NT - 1)
        def _():
            _emit_chunk(4, _TAIL_HI)


def _sc_scatter(P_flat, idx1, idx2):
    mesh = plsc.VectorSubcoreMesh(core_axis_name="c", subcore_axis_name="s",
                                  num_cores=_NC, num_subcores=_NT)
    fn = functools.partial(
        pl.kernel,
        out_type=jax.ShapeDtypeStruct((4 * _N * 20,), jnp.float32),
        mesh=mesh,
        compiler_params=pltpu.CompilerParams(needs_layout_passes=False),
        scratch_types=[
            pltpu.VMEM((_W * _PS,), jnp.float32),
            pltpu.VMEM((_ACCW,), jnp.float32),
            pltpu.VMEM((_W,), jnp.int32),
            pltpu.VMEM((_W,), jnp.int32),
        ],
    )(_sc_body)
    return fn(P_flat, idx1, idx2)


def _finish_body(a, hns, hnv, hnt, wns1, bns1, wns2, bns2, wnv1,
                 wnt1, wc1ns, wc1nv, wc1nt, bc1,
                 wc2ns, bc2ns, wc2nv, bc2nv, wc2nt, bc2nt, wnv2, wnt2,
                 ns_out, nv_out, nt_out):
    af = jnp.concatenate([a[0], a[1], a[2], a[3]], axis=-1)
    sd = _silu(jnp.dot(hns[...], wns1[...]) + bns1[...])
    sd = jnp.dot(sd, wns2[...]) + bns2[...]
    v1 = jnp.dot(hnv[...], wnv1[...])
    ns_int = af[:, :32] + sd
    nv_int = af[:, 32:] + v1
    sq = nv_int * nv_int
    nv_norm = jnp.sqrt(sq[:, 0:16] + sq[:, 16:32] + sq[:, 32:48])
    nt1 = jnp.dot(hnt[...], wnt1[...])
    sqt = nt1 * nt1
    acc = sqt[:, 0:16]
    for k in range(1, 9):
        acc = acc + sqt[:, 16 * k:16 * (k + 1)]
    nt_norm = jnp.sqrt(acc)
    h1 = _silu(jnp.dot(ns_int, wc1ns[...]) + jnp.dot(nv_norm, wc1nv[...])
               + jnp.dot(nt_norm, wc1nt[...]) + bc1[...])
    ns_out[...] = jnp.dot(h1, wc2ns[...]) + bc2ns[...] + hns[...]
    gv = _sigmoid(jnp.dot(h1, wc2nv[...]) + bc2nv[...])
    gt = _sigmoid(jnp.dot(h1, wc2nt[...]) + bc2nt[...])
    nv2 = jnp.dot(hnv[...], wnv2[...])
    nv_out[...] = jnp.concatenate([gv, gv, gv], axis=-1) * nv2 + hnv[...]
    nt2 = jnp.dot(hnt[...], wnt2[...])
    gt9 = jnp.concatenate([gt] * 9, axis=-1)
    nt_out[...] = gt9 * nt2 + hnt[...]


def _node_finish(A, h_ns, h_nv_flat, h_nt_flat, Wns1T, bns1, Wns2T, bns2,
                 Wnv1kT, Wnt1kT, Wc1Tns, Wc1Tnv,
                 Wc1Tnt, bc1, Wc2Tns, bc2ns, Wc2Tnv, bc2nv, Wc2Tnt, bc2nt,
                 Wnv2kT, Wnt2kT):
    grid = (_N // _BN,)
    full = lambda shape: pl.BlockSpec(shape, lambda i: tuple(0 for _ in shape))
    return pl.pallas_call(
        _finish_body,
        grid=grid,
        in_specs=[
            pl.BlockSpec((4, _BN, 20), lambda i: (0, i, 0)),
            pl.BlockSpec((_BN, 64), lambda i: (i, 0)),
            pl.BlockSpec((_BN, 48), lambda i: (i, 0)),
            pl.BlockSpec((_BN, 144), lambda i: (i, 0)),
            full((64, 32)),
            full((1, 32)),
            full((32, 32)),
            full((1, 32)),
            full((48, 48)),
            full((144, 144)),
            full((32, 32)),
            full((16, 32)),
            full((16, 32)),
            full((1, 32)),
            full((32, 64)),
            full((1, 64)),
            full((32, 16)),
            full((1, 16)),
            full((32, 16)),
            full((1, 16)),
            full((48, 48)),
            full((144, 144)),
        ],
        out_specs=[
            pl.BlockSpec((_BN, 64), lambda i: (i, 0)),
            pl.BlockSpec((_BN, 48), lambda i: (i, 0)),
            pl.BlockSpec((_BN, 144), lambda i: (i, 0)),
        ],
        out_shape=[
            jax.ShapeDtypeStruct((_N, 64), jnp.float32),
            jax.ShapeDtypeStruct((_N, 48), jnp.float32),
            jax.ShapeDtypeStruct((_N, 144), jnp.float32),
        ],
        compiler_params=pltpu.CompilerParams(
            dimension_semantics=("arbitrary",)),
    )(A, h_ns, h_nv_flat, h_nt_flat, Wns1T, bns1, Wns2T, bns2, Wnv1kT,
      Wnt1kT, Wc1Tns, Wc1Tnv, Wc1Tnt, bc1,
      Wc2Tns, bc2ns, Wc2Tnv, bc2nv, Wc2Tnt, bc2nt, Wnv2kT, Wnt2kT)


def kernel(h_ns, h_nv, h_nt, h_es, h_ev, atom_index1, atom_index2,
           Wns1, bns1, Wns2, bns2, Wes1, bes1, Wes2, bes2,
           Wnv1, Wnv2, Wnt1, Wnt2, Wev1, Wc1, bc1, Wc2, bc2):
    f32 = jnp.float32
    h_ev_flat = h_ev.reshape(_E, 48)
    h_nv_flat = h_nv.reshape(_N, 48)
    h_nt_flat = h_nt.reshape(_N, 144)
    i3 = jnp.eye(3, dtype=f32)
    i9 = jnp.eye(9, dtype=f32)
    WevkT = jnp.kron(i3, Wev1).T
    Wnv1kT = jnp.kron(i3, Wnv1).T
    Wnv2kT = jnp.kron(i3, Wnv2).T
    Wnt1kT = jnp.kron(i9, Wnt1).T
    Wnt2kT = jnp.kron(i9, Wnt2).T
    Wc1T = Wc1.T
    Wc2T = Wc2.T

    P = _edge_payload(h_es, h_ev_flat, Wes1.T, bes1.reshape(1, 32),
                      Wes2.T, bes2.reshape(1, 32), WevkT)
    idx1 = atom_index1.astype(jnp.int32)
    idx2 = atom_index2.astype(jnp.int32)
    A_flat = _sc_scatter(P.reshape(4 * _E * _PS), idx1, idx2)
    A = A_flat.reshape(4, _N, 20)
    ns_out, nv_out_flat, nt_out_flat = _node_finish(
        A, h_ns, h_nv_flat, h_nt_flat, Wns1.T, bns1.reshape(1, 32),
        Wns2.T, bns2.reshape(1, 32), Wnv1kT, Wnt1kT,
        Wc1T[0:32], Wc1T[32:48], Wc1T[48:64], bc1.reshape(1, 32),
        Wc2T[:, 0:64], bc2[0:64].reshape(1, 64),
        Wc2T[:, 64:80], bc2[64:80].reshape(1, 16),
        Wc2T[:, 80:96], bc2[80:96].reshape(1, 16),
        Wnv2kT, Wnt2kT)
    return (ns_out, nv_out_flat.reshape(_N, 3, 16),
            nt_out_flat.reshape(_N, 3, 3, 16))


# speedup vs baseline: 8.1471x; 1.2041x over previous
import functools

import jax
import jax.numpy as jnp
from jax import lax
from jax.experimental import pallas as pl
from jax.experimental.pallas import tpu as pltpu
from jax.experimental.pallas import tpu_sc as plsc

_N = 50000
_E = 800000

_NC = 2
_NT = 16
_W = 640
_NWIN = _E // _W
_RPT = 3128
_PS = 21
_ACCW = _RPT * _PS + 16
_ZITER = _ACCW // 16
_WCH = 640
_TAIL_LO = _RPT - 4 * _WCH
_TAIL_HI = (_N - 15 * _RPT) - 4 * _WCH

_BE = 8000
_BN = 2000


def _silu(x):
    return x * (1.0 / (1.0 + jnp.exp(-x)))


def _sigmoid(x):
    return 1.0 / (1.0 + jnp.exp(-x))


def _edge_body(hes, hev, wes1, bes1, wes2, bes2, wev, out):
    s = _silu(jnp.dot(hes[...], wes1[...]) + bes1[...])
    s = jnp.dot(s, wes2[...]) + bes2[...]
    v = jnp.dot(hev[...], wev[...])
    z = jnp.zeros_like(s[:, 0:1])
    out[0] = jnp.concatenate([s[:, 0:20], z], axis=-1)
    out[1] = jnp.concatenate([s[:, 20:32], v[:, 0:8], z], axis=-1)
    out[2] = jnp.concatenate([v[:, 8:28], z], axis=-1)
    out[3] = jnp.concatenate([v[:, 28:48], z], axis=-1)


def _edge_payload(h_es, h_ev_flat, Wes1T, bes1, Wes2T, bes2, WevkT):
    grid = (_E // _BE,)
    return pl.pallas_call(
        _edge_body,
        grid=grid,
        in_specs=[
            pl.BlockSpec((_BE, 64), lambda i: (i, 0)),
            pl.BlockSpec((_BE, 48), lambda i: (i, 0)),
            pl.BlockSpec((64, 32), lambda i: (0, 0)),
            pl.BlockSpec((1, 32), lambda i: (0, 0)),
            pl.BlockSpec((32, 32), lambda i: (0, 0)),
            pl.BlockSpec((1, 32), lambda i: (0, 0)),
            pl.BlockSpec((48, 48), lambda i: (0, 0)),
        ],
        out_specs=pl.BlockSpec((4, _BE, _PS), lambda i: (0, i, 0)),
        out_shape=jax.ShapeDtypeStruct((4, _E, _PS), jnp.float32),
        compiler_params=pltpu.CompilerParams(
            dimension_semantics=("arbitrary",)),
    )(h_es, h_ev_flat, Wes1T, bes1, Wes2T, bes2, WevkT)


def _sc_body(P, idx1, idx2, A, pbuf, acc, ib1, ib2):
    c = lax.axis_index("c")
    s = lax.axis_index("s")
    lane = lax.iota(jnp.int32, 16)
    zero16 = jnp.zeros((16,), jnp.float32)
    lo = s * _RPT
    hi = jnp.minimum(lo + _RPT, _N)

    for k in range(2):
        p = 2 * k + c

        def zero_body(i, cc):
            acc[pl.ds(i * 16, 16)] = zero16
            return cc

        lax.fori_loop(0, _ZITER, zero_body, 0)

        def win_body(w, carry):
            e0 = w * _W
            pltpu.sync_copy(
                P.at[pl.ds(p * (_E * _PS) + e0 * _PS, _W * _PS)], pbuf)
            pltpu.sync_copy(idx1.at[pl.ds(e0, _W)], ib1)
            pltpu.sync_copy(idx2.at[pl.ds(e0, _W)], ib2)

            @plsc.parallel_loop(0, _W // 16, unroll=4)
            def grp_body(g):
                eoff = (g * 16 + lane) * _PS
                for ib in (ib1, ib2):
                    iv = ib[pl.ds(g * 16, 16)]
                    m = (iv >= lo) & (iv < hi)
                    base = jnp.where(m, (iv - lo) * _PS, 0)
                    for col in range(20):
                        vals = plsc.load_gather(pbuf, [eoff + col])
                        plsc.addupdate_scatter(acc, [base + col], vals,
                                               mask=m)

            return carry

        lax.fori_loop(0, _NWIN, win_body, 0)

        def _emit_chunk(j, rows):
            def rs_body(i, cc):
                v = i * 16 + lane
                q = v // 20
                r = v - q * 20
                vals = plsc.load_gather(acc, [(j * _WCH + q) * _PS + r])
                pbuf[pl.ds(i * 16, 16)] = vals
                return cc

            lax.fori_loop(0, rows * 20 // 16, rs_body, 0)
            pltpu.sync_copy(
                pbuf.at[pl.ds(0, rows * 20)],
                A.at[pl.ds(p * (_N * 20) + (lo + j * _WCH) * 20, rows * 20)])

        for j in range(4):
            _emit_chunk(j, _WCH)

        @pl.when(s < _NT - 1)
        def _():
            _emit_chunk(4, _TAIL_LO)

        @pl.when(s == _NT - 1)
        def _():
            _emit_chunk(4, _TAIL_HI)


def _sc_scatter(P_flat, idx1, idx2):
    mesh = plsc.VectorSubcoreMesh(core_axis_name="c", subcore_axis_name="s",
                                  num_cores=_NC, num_subcores=_NT)
    fn = functools.partial(
        pl.kernel,
        out_type=jax.ShapeDtypeStruct((4 * _N * 20,), jnp.float32),
        mesh=mesh,
        compiler_params=pltpu.CompilerParams(needs_layout_passes=False),
        scratch_types=[
            pltpu.VMEM((_W * _PS,), jnp.float32),
            pltpu.VMEM((_ACCW,), jnp.float32),
            pltpu.VMEM((_W,), jnp.int32),
            pltpu.VMEM((_W,), jnp.int32),
        ],
    )(_sc_body)
    return fn(P_flat, idx1, idx2)


def _finish_body(a, hns, hnv, hnt, wns1, bns1, wns2, bns2, wnv1,
                 wnt1, wc1ns, wc1nv, wc1nt, bc1,
                 wc2ns, bc2ns, wc2nv, bc2nv, wc2nt, bc2nt, wnv2, wnt2,
                 ns_out, nv_out, nt_out):
    af = jnp.concatenate([a[0], a[1], a[2], a[3]], axis=-1)
    sd = _silu(jnp.dot(hns[...], wns1[...]) + bns1[...])
    sd = jnp.dot(sd, wns2[...]) + bns2[...]
    v1 = jnp.dot(hnv[...], wnv1[...])
    ns_int = af[:, :32] + sd
    nv_int = af[:, 32:] + v1
    sq = nv_int * nv_int
    nv_norm = jnp.sqrt(sq[:, 0:16] + sq[:, 16:32] + sq[:, 32:48])
    nt1 = jnp.dot(hnt[...], wnt1[...])
    sqt = nt1 * nt1
    acc = sqt[:, 0:16]
    for k in range(1, 9):
        acc = acc + sqt[:, 16 * k:16 * (k + 1)]
    nt_norm = jnp.sqrt(acc)
    h1 = _silu(jnp.dot(ns_int, wc1ns[...]) + jnp.dot(nv_norm, wc1nv[...])
               + jnp.dot(nt_norm, wc1nt[...]) + bc1[...])
    ns_out[...] = jnp.dot(h1, wc2ns[...]) + bc2ns[...] + hns[...]
    gv = _sigmoid(jnp.dot(h1, wc2nv[...]) + bc2nv[...])
    gt = _sigmoid(jnp.dot(h1, wc2nt[...]) + bc2nt[...])
    nv2 = jnp.dot(hnv[...], wnv2[...])
    nv_out[...] = jnp.concatenate([gv, gv, gv], axis=-1) * nv2 + hnv[...]
    nt2 = jnp.dot(hnt[...], wnt2[...])
    gt9 = jnp.concatenate([gt] * 9, axis=-1)
    nt_out[...] = gt9 * nt2 + hnt[...]


def _node_finish(A, h_ns, h_nv_flat, h_nt_flat, Wns1T, bns1, Wns2T, bns2,
                 Wnv1kT, Wnt1kT, Wc1Tns, Wc1Tnv,
                 Wc1Tnt, bc1, Wc2Tns, bc2ns, Wc2Tnv, bc2nv, Wc2Tnt, bc2nt,
                 Wnv2kT, Wnt2kT):
    grid = (_N // _BN,)
    full = lambda shape: pl.BlockSpec(shape, lambda i: tuple(0 for _ in shape))
    return pl.pallas_call(
        _finish_body,
        grid=grid,
        in_specs=[
            pl.BlockSpec((4, _BN, 20), lambda i: (0, i, 0)),
            pl.BlockSpec((_BN, 64), lambda i: (i, 0)),
            pl.BlockSpec((_BN, 48), lambda i: (i, 0)),
            pl.BlockSpec((_BN, 144), lambda i: (i, 0)),
            full((64, 32)),
            full((1, 32)),
            full((32, 32)),
            full((1, 32)),
            full((48, 48)),
            full((144, 144)),
            full((32, 32)),
            full((16, 32)),
            full((16, 32)),
            full((1, 32)),
            full((32, 64)),
            full((1, 64)),
            full((32, 16)),
            full((1, 16)),
            full((32, 16)),
            full((1, 16)),
            full((48, 48)),
            full((144, 144)),
        ],
        out_specs=[
            pl.BlockSpec((_BN, 64), lambda i: (i, 0)),
            pl.BlockSpec((_BN, 48), lambda i: (i, 0)),
            pl.BlockSpec((_BN, 144), lambda i: (i, 0)),
        ],
        out_shape=[
            jax.ShapeDtypeStruct((_N, 64), jnp.float32),
            jax.ShapeDtypeStruct((_N, 48), jnp.float32),
            jax.ShapeDtypeStruct((_N, 144), jnp.float32),
        ],
        compiler_params=pltpu.CompilerParams(
            dimension_semantics=("arbitrary",)),
    )(A, h_ns, h_nv_flat, h_nt_flat, Wns1T, bns1, Wns2T, bns2, Wnv1kT,
      Wnt1kT, Wc1Tns, Wc1Tnv, Wc1Tnt, bc1,
      Wc2Tns, bc2ns, Wc2Tnv, bc2nv, Wc2Tnt, bc2nt, Wnv2kT, Wnt2kT)


def kernel(h_ns, h_nv, h_nt, h_es, h_ev, atom_index1, atom_index2,
           Wns1, bns1, Wns2, bns2, Wes1, bes1, Wes2, bes2,
           Wnv1, Wnv2, Wnt1, Wnt2, Wev1, Wc1, bc1, Wc2, bc2):
    f32 = jnp.float32
    h_ev_flat = h_ev.reshape(_E, 48)
    h_nv_flat = h_nv.reshape(_N, 48)
    h_nt_flat = h_nt.reshape(_N, 144)
    i3 = jnp.eye(3, dtype=f32)
    i9 = jnp.eye(9, dtype=f32)
    WevkT = jnp.kron(i3, Wev1).T
    Wnv1kT = jnp.kron(i3, Wnv1).T
    Wnv2kT = jnp.kron(i3, Wnv2).T
    Wnt1kT = jnp.kron(i9, Wnt1).T
    Wnt2kT = jnp.kron(i9, Wnt2).T
    Wc1T = Wc1.T
    Wc2T = Wc2.T

    P = _edge_payload(h_es, h_ev_flat, Wes1.T, bes1.reshape(1, 32),
                      Wes2.T, bes2.reshape(1, 32), WevkT)
    idx1 = atom_index1.astype(jnp.int32)
    idx2 = atom_index2.astype(jnp.int32)
    A_flat = _sc_scatter(P.reshape(4 * _E * _PS), idx1, idx2)
    A = A_flat.reshape(4, _N, 20)
    ns_out, nv_out_flat, nt_out_flat = _node_finish(
        A, h_ns, h_nv_flat, h_nt_flat, Wns1.T, bns1.reshape(1, 32),
        Wns2.T, bns2.reshape(1, 32), Wnv1kT, Wnt1kT,
        Wc1T[0:32], Wc1T[32:48], Wc1T[48:64], bc1.reshape(1, 32),
        Wc2T[:, 0:64], bc2[0:64].reshape(1, 64),
        Wc2T[:, 64:80], bc2[64:80].reshape(1, 16),
        Wc2T[:, 80:96], bc2[80:96].reshape(1, 16),
        Wnv2kT, Wnt2kT)
    return (ns_out, nv_out_flat.reshape(_N, 3, 16),
            nt_out_flat.reshape(_N, 3, 3, 16))
